# Initial kernel scaffold; baseline (speedup 1.0000x reference)
import jax
import jax.numpy as jnp
from jax.experimental import pallas as pl


def kernel(x, history, history_len, supp_users, edge_sparse, user_embedding, item_embedding, wq, wk, wv, w_out_W, gcn_user_W, gcn_user_b, gcn_item_W, gcn_item_b, l1_W, l1_b, l2_W, l2_b, l3_W, l3_b, user_bias, item_bias):
    raise NotImplementedError("write your pallas kernel here")


# speedup vs baseline: 9.1246x; 9.1246x over previous
import functools

import jax
import jax.numpy as jnp
from jax import lax
from jax.experimental import pallas as pl
from jax.experimental.pallas import tpu as pltpu
from jax.experimental.pallas import tpu_sc as plsc

B = 4096
EMB = 64
HEADS = 4
SAMPLE = 100
HIST = 50
SUPP = 10000
E = 1000000
NU = 100000
NI = 100000
NC = 2
NS = 16
NW = NC * NS

INV_N = 100352
INV_PT = INV_N // NS
HIST_P = 56
SUPP_P = 10240
NEIGH_N = HEADS * B * SAMPLE
EPAD = 1024000
ECHUNK = 2000
CHUNKS_PER_TILE = EPAD // ECHUNK // NW
Q = 512
PEND = 2560
ACC_R = 4160
ACC_PT = ACC_R // NS
TRASH = 4096

def _wid():
    return lax.axis_index("s") * NC + lax.axis_index("c")


def _inv_body(uid_hbm, iid_hbm, invu_hbm, invi_hbm, neg_v, idx_v, val_v, sem):
    cid = lax.axis_index("c")
    sid = lax.axis_index("s")

    @pl.when(cid == 0)
    def _():
        def fill(i, _):
            neg_v[pl.ds(i * 16, 16)] = jnp.full((16,), -1, jnp.int32)
            return 0
        lax.fori_loop(0, INV_PT // 16, fill, 0)
        base = sid * INV_PT
        pltpu.sync_copy(neg_v, invu_hbm.at[pl.ds(base, INV_PT)])
        pltpu.sync_copy(neg_v, invi_hbm.at[pl.ds(base, INV_PT)])
        plsc.subcore_barrier()
        b0 = sid * (B // NS)
        lane = lax.iota(jnp.int32, 16)

        def fillv(i, _):
            val_v[pl.ds(i * 16, 16)] = b0 + i * 16 + lane
            return 0
        lax.fori_loop(0, (B // NS) // 16, fillv, 0)
        pltpu.sync_copy(uid_hbm.at[pl.ds(b0, B // NS)], idx_v)
        pltpu.async_copy(val_v, invu_hbm.at[idx_v], sem).wait()
        pltpu.sync_copy(iid_hbm.at[pl.ds(b0, B // NS)], idx_v)
        pltpu.async_copy(val_v, invi_hbm.at[idx_v], sem).wait()


def _mk_inv(mesh):
    return functools.partial(
        pl.kernel,
        out_type=(jax.ShapeDtypeStruct((INV_N,), jnp.int32),
                  jax.ShapeDtypeStruct((INV_N,), jnp.int32)),
        mesh=mesh,
        compiler_params=pltpu.CompilerParams(use_tc_tiling_on_sc=False, needs_layout_passes=False),
        scratch_types=[
            pltpu.VMEM((INV_PT,), jnp.int32),
            pltpu.VMEM((B // NS,), jnp.int32),
            pltpu.VMEM((B // NS,), jnp.int32),
            pltpu.SemaphoreType.DMA,
        ],
    )(_inv_body)


def _supp_body(supp_hbm, uemb_hbm, out_hbm, idx_v, rows_v, sem):
    base = _wid() * (SUPP_P // NW)
    pltpu.sync_copy(supp_hbm.at[pl.ds(base, SUPP_P // NW)], idx_v)
    pltpu.async_copy(uemb_hbm.at[idx_v], rows_v, sem).wait()
    pltpu.sync_copy(rows_v, out_hbm.at[pl.ds(base, SUPP_P // NW), :])


def _mk_supp(mesh):
    return functools.partial(
        pl.kernel,
        out_type=jax.ShapeDtypeStruct((SUPP_P, EMB), jnp.float32),
        mesh=mesh,
        compiler_params=pltpu.CompilerParams(use_tc_tiling_on_sc=False, needs_layout_passes=False),
        scratch_types=[
            pltpu.VMEM((SUPP_P // NW,), jnp.int32),
            pltpu.VMEM((SUPP_P // NW, EMB), jnp.float32),
            pltpu.SemaphoreType.DMA,
        ],
    )(_supp_body)


_NPT = NEIGH_N // NW
_NCK = 512


def _gather_body(si_hbm, hist_hbm, iid_hbm, esupp_hbm, iemb_hbm,
                 neigh_hbm, hsum_hbm, irows_hbm,
                 idx_v, rows_v, iidx_v, irow_v, hidx_v, hrows_v, hout_v, sem):
    w = _wid()
    nbase = w * _NPT

    def chunk(i, _):
        off = nbase + i * _NCK
        pltpu.sync_copy(si_hbm.at[pl.ds(off, _NCK)], idx_v)
        pltpu.async_copy(esupp_hbm.at[idx_v], rows_v, sem).wait()
        pltpu.sync_copy(rows_v, neigh_hbm.at[pl.ds(off, _NCK), :])
        return 0
    lax.fori_loop(0, _NPT // _NCK, chunk, 0)

    ibase = w * (B // NW)
    pltpu.sync_copy(iid_hbm.at[pl.ds(ibase, B // NW)], iidx_v)
    pltpu.async_copy(iemb_hbm.at[iidx_v], irow_v, sem).wait()
    pltpu.sync_copy(irow_v, irows_hbm.at[pl.ds(ibase, B // NW), :])

    def hb(j, _):
        b = ibase + j
        pltpu.sync_copy(hist_hbm.at[b], hidx_v)
        pltpu.async_copy(iemb_hbm.at[hidx_v], hrows_v, sem).wait()
        z = jnp.zeros((16,), jnp.float32)

        def srow(k, acc):
            return tuple(acc[t] + hrows_v[k, pl.ds(t * 16, 16)] for t in range(4))
        acc = lax.fori_loop(0, HIST, srow, (z, z, z, z))
        for t in range(4):
            hout_v[pl.ds(t * 16, 16)] = acc[t]
        pltpu.sync_copy(hout_v, hsum_hbm.at[b])
        return 0
    lax.fori_loop(0, B // NW, hb, 0)


def _mk_gather(mesh):
    return functools.partial(
        pl.kernel,
        out_type=(jax.ShapeDtypeStruct((NEIGH_N, EMB), jnp.float32),
                  jax.ShapeDtypeStruct((B, EMB), jnp.float32),
                  jax.ShapeDtypeStruct((B, EMB), jnp.float32)),
        mesh=mesh,
        compiler_params=pltpu.CompilerParams(use_tc_tiling_on_sc=False, needs_layout_passes=False),
        scratch_types=[
            pltpu.VMEM((_NCK,), jnp.int32),
            pltpu.VMEM((_NCK, EMB), jnp.float32),
            pltpu.VMEM((B // NW,), jnp.int32),
            pltpu.VMEM((B // NW, EMB), jnp.float32),
            pltpu.VMEM((HIST_P,), jnp.int32),
            pltpu.VMEM((HIST_P, EMB), jnp.float32),
            pltpu.VMEM((EMB,), jnp.float32),
            pltpu.SemaphoreType.DMA,
        ],
    )(_gather_body)


RING = 8
_QSH = 9
_RMSK = RING * Q - 1


def _flush_row(t2, d2, table_hbm, acc_s, cnt_s, rows_v, ones_v, sem, f):
    rf = (f >> _QSH) & (RING - 1)
    pltpu.async_copy(table_hbm.at[d2.at[rf]], rows_v, sem).wait()
    pltpu.sync_copy(rows_v, acc_s.at[t2.at[rf]], add=True)
    pltpu.sync_copy(ones_v, cnt_s.at[t2.at[rf]], add=True)


def _drain(t2, d2, table_hbm, acc_s, cnt_s, rows_v, ones_v, sem, pu, f):
    nf = (pu - f) >> _QSH

    def fl(j, _):
        _flush_row(t2, d2, table_hbm, acc_s, cnt_s, rows_v, ones_v, sem,
                   f + j * Q)
        return 0
    lax.fori_loop(0, nf, fl, 0)
    return f + nf * Q


def _pad_tail(t2, d2, pu, f):
    lane = lax.iota(jnp.int32, 16)

    def pj(j, _):
        idx = f + j * 16 + lane
        m = idx >= pu
        r = idx & _RMSK
        plsc.store_scatter(t2, [r >> _QSH, r & (Q - 1)],
                           jnp.full((16,), TRASH, jnp.int32), mask=m)
        plsc.store_scatter(d2, [r >> _QSH, r & (Q - 1)],
                           jnp.zeros((16,), jnp.int32), mask=m)
        return 0
    lax.fori_loop(0, Q // 16, pj, 0)


def _edge_body(src_hbm, dst_hbm, invu_hbm, invi_hbm, iemb_hbm, uemb_hbm,
               accu_hbm, acci_hbm, cntu_hbm, cnti_hbm,
               s_v, d_v, tu_v, ti_v,
               pu_t, pu_d, pi_t, pi_s,
               rows_v, ones_v, zbuf, zcnt,
               accu_s, acci_s, cntu_s, cnti_s, sem):
    cid = lax.axis_index("c")
    sid = lax.axis_index("s")
    w = sid * NC + cid
    lane = lax.iota(jnp.int32, 16)
    onerow = jnp.where(lane == 0, 1.0, 0.0).astype(jnp.float32)

    def f1(i, _):
        ones_v[i, pl.ds(0, 16)] = onerow
        return 0
    lax.fori_loop(0, Q, f1, 0)

    zv = jnp.zeros((16,), jnp.float32)

    def fz(i, _):
        for t in range(4):
            zbuf[i, pl.ds(t * 16, 16)] = zv
        zcnt[i, pl.ds(0, 16)] = zv
        return 0
    lax.fori_loop(0, ACC_PT, fz, 0)

    rb = sid * ACC_PT
    pltpu.sync_copy(zbuf, accu_s.at[pl.ds(rb, ACC_PT), :])
    pltpu.sync_copy(zbuf, acci_s.at[pl.ds(rb, ACC_PT), :])
    pltpu.sync_copy(zcnt, cntu_s.at[pl.ds(rb, ACC_PT), :])
    pltpu.sync_copy(zcnt, cnti_s.at[pl.ds(rb, ACC_PT), :])
    plsc.subcore_barrier()

    def chunk_fn(c, carry):
        pu, fu, pi, fi = carry
        off = (w * CHUNKS_PER_TILE + c) * ECHUNK
        pltpu.sync_copy(src_hbm.at[pl.ds(off, ECHUNK)], s_v)
        pltpu.sync_copy(dst_hbm.at[pl.ds(off, ECHUNK)], d_v)
        pltpu.async_copy(invu_hbm.at[s_v], tu_v, sem).wait()
        pltpu.async_copy(invi_hbm.at[d_v], ti_v, sem).wait()

        def grp(g, pp):
            pu, pi = pp
            s16 = s_v[pl.ds(g * 16, 16)]
            d16 = d_v[pl.ds(g * 16, 16)]
            tu16 = tu_v[pl.ds(g * 16, 16)]
            ti16 = ti_v[pl.ds(g * 16, 16)]
            mu = tu16 >= 0
            mi = ti16 >= 0
            cu = plsc.cumsum(mu.astype(jnp.int32))
            iu = (pu + cu - 1) & _RMSK
            plsc.store_scatter(pu_t, [iu >> _QSH, iu & (Q - 1)], tu16, mask=mu)
            plsc.store_scatter(pu_d, [iu >> _QSH, iu & (Q - 1)], d16, mask=mu)
            pu = pu + jnp.sum(mu.astype(jnp.int32))
            ci = plsc.cumsum(mi.astype(jnp.int32))
            ii = (pi + ci - 1) & _RMSK
            plsc.store_scatter(pi_t, [ii >> _QSH, ii & (Q - 1)], ti16, mask=mi)
            plsc.store_scatter(pi_s, [ii >> _QSH, ii & (Q - 1)], s16, mask=mi)
            pi = pi + jnp.sum(mi.astype(jnp.int32))
            return (pu, pi)
        pu, pi = lax.fori_loop(0, ECHUNK // 16, grp, (pu, pi))
        fu = _drain(pu_t, pu_d, iemb_hbm, accu_s, cntu_s, rows_v, ones_v,
                    sem, pu, fu)
        fi = _drain(pi_t, pi_s, uemb_hbm, acci_s, cnti_s, rows_v, ones_v,
                    sem, pi, fi)
        return (pu, fu, pi, fi)

    z32 = jnp.int32(0)
    pu, fu, pi, fi = lax.fori_loop(0, CHUNKS_PER_TILE, chunk_fn,
                                   (z32, z32, z32, z32))

    _pad_tail(pu_t, pu_d, pu, fu)
    _flush_row(pu_t, pu_d, iemb_hbm, accu_s, cntu_s, rows_v, ones_v, sem, fu)
    _pad_tail(pi_t, pi_s, pi, fi)
    _flush_row(pi_t, pi_s, uemb_hbm, acci_s, cnti_s, rows_v, ones_v, sem, fi)

    plsc.subcore_barrier()
    pltpu.sync_copy(accu_s.at[pl.ds(rb, ACC_PT), :],
                    accu_hbm.at[cid, pl.ds(rb, ACC_PT), :])
    pltpu.sync_copy(acci_s.at[pl.ds(rb, ACC_PT), :],
                    acci_hbm.at[cid, pl.ds(rb, ACC_PT), :])
    pltpu.sync_copy(cntu_s.at[pl.ds(rb, ACC_PT), :],
                    cntu_hbm.at[cid, pl.ds(rb, ACC_PT), :])
    pltpu.sync_copy(cnti_s.at[pl.ds(rb, ACC_PT), :],
                    cnti_hbm.at[cid, pl.ds(rb, ACC_PT), :])


def _mk_edge(mesh):
    return functools.partial(
        pl.kernel,
        out_type=(jax.ShapeDtypeStruct((NC, ACC_R, EMB), jnp.float32),
                  jax.ShapeDtypeStruct((NC, ACC_R, EMB), jnp.float32),
                  jax.ShapeDtypeStruct((NC, ACC_R, 16), jnp.float32),
                  jax.ShapeDtypeStruct((NC, ACC_R, 16), jnp.float32)),
        mesh=mesh,
        compiler_params=pltpu.CompilerParams(use_tc_tiling_on_sc=False, needs_layout_passes=False),
        scratch_types=[
            pltpu.VMEM((ECHUNK,), jnp.int32),
            pltpu.VMEM((ECHUNK,), jnp.int32),
            pltpu.VMEM((ECHUNK,), jnp.int32),
            pltpu.VMEM((ECHUNK,), jnp.int32),
            pltpu.VMEM((RING, Q), jnp.int32),
            pltpu.VMEM((RING, Q), jnp.int32),
            pltpu.VMEM((RING, Q), jnp.int32),
            pltpu.VMEM((RING, Q), jnp.int32),
            pltpu.VMEM((Q, EMB), jnp.float32),
            pltpu.VMEM((Q, 16), jnp.float32),
            pltpu.VMEM((ACC_PT, EMB), jnp.float32),
            pltpu.VMEM((ACC_PT, 16), jnp.float32),
            pltpu.VMEM_SHARED((ACC_R, EMB), jnp.float32),
            pltpu.VMEM_SHARED((ACC_R, EMB), jnp.float32),
            pltpu.VMEM_SHARED((ACC_R, 16), jnp.float32),
            pltpu.VMEM_SHARED((ACC_R, 16), jnp.float32),
            pltpu.SemaphoreType.DMA,
        ],
    )(_edge_body)


_BPT = B // NW


def _bg_body(uid_hbm, iid_hbm, invu_hbm, invi_hbm,
             au0, au1, ai0, ai1, cu0, cu1, ci0, ci1, ub_hbm, ib_hbm,
             gi_hbm, gu_hbm, du_hbm, di_hbm, bu_hbm, bi_hbm,
             uidv, iidv, tbu, tbi, r0, r1, c0, c1, bv, sem):
    base = _wid() * _BPT
    pltpu.sync_copy(uid_hbm.at[pl.ds(base, _BPT)], uidv)
    pltpu.sync_copy(iid_hbm.at[pl.ds(base, _BPT)], iidv)
    pltpu.async_copy(invu_hbm.at[uidv], tbu, sem).wait()
    pltpu.async_copy(invi_hbm.at[iidv], tbi, sem).wait()

    def addrows(j, _):
        for t in range(4):
            a = r0[j, pl.ds(t * 16, 16)]
            b = r1[j, pl.ds(t * 16, 16)]
            r0[j, pl.ds(t * 16, 16)] = a + b
        return 0

    def addcnt(j, _):
        a = c0[j, pl.ds(0, 16)]
        b = c1[j, pl.ds(0, 16)]
        c0[j, pl.ds(0, 16)] = a + b
        return 0

    pltpu.async_copy(au0.at[tbu], r0, sem).wait()
    pltpu.async_copy(au1.at[tbu], r1, sem).wait()
    lax.fori_loop(0, _BPT, addrows, 0)
    pltpu.sync_copy(r0, gi_hbm.at[pl.ds(base, _BPT), :])

    pltpu.async_copy(ai0.at[tbi], r0, sem).wait()
    pltpu.async_copy(ai1.at[tbi], r1, sem).wait()
    lax.fori_loop(0, _BPT, addrows, 0)
    pltpu.sync_copy(r0, gu_hbm.at[pl.ds(base, _BPT), :])

    pltpu.async_copy(cu0.at[tbu], c0, sem).wait()
    pltpu.async_copy(cu1.at[tbu], c1, sem).wait()
    lax.fori_loop(0, _BPT, addcnt, 0)
    pltpu.sync_copy(c0, du_hbm.at[pl.ds(base, _BPT), :])

    pltpu.async_copy(ci0.at[tbi], c0, sem).wait()
    pltpu.async_copy(ci1.at[tbi], c1, sem).wait()
    lax.fori_loop(0, _BPT, addcnt, 0)
    pltpu.sync_copy(c0, di_hbm.at[pl.ds(base, _BPT), :])

    pltpu.async_copy(ub_hbm.at[uidv], bv, sem).wait()
    pltpu.sync_copy(bv, bu_hbm.at[pl.ds(base, _BPT)])
    pltpu.async_copy(ib_hbm.at[iidv], bv, sem).wait()
    pltpu.sync_copy(bv, bi_hbm.at[pl.ds(base, _BPT)])


def _mk_bg(mesh):
    return functools.partial(
        pl.kernel,
        out_type=(jax.ShapeDtypeStruct((B, EMB), jnp.float32),
                  jax.ShapeDtypeStruct((B, EMB), jnp.float32),
                  jax.ShapeDtypeStruct((B, 16), jnp.float32),
                  jax.ShapeDtypeStruct((B, 16), jnp.float32),
                  jax.ShapeDtypeStruct((B,), jnp.float32),
                  jax.ShapeDtypeStruct((B,), jnp.float32)),
        mesh=mesh,
        compiler_params=pltpu.CompilerParams(use_tc_tiling_on_sc=False, needs_layout_passes=False),
        scratch_types=[
            pltpu.VMEM((_BPT,), jnp.int32),
            pltpu.VMEM((_BPT,), jnp.int32),
            pltpu.VMEM((_BPT,), jnp.int32),
            pltpu.VMEM((_BPT,), jnp.int32),
            pltpu.VMEM((_BPT, EMB), jnp.float32),
            pltpu.VMEM((_BPT, EMB), jnp.float32),
            pltpu.VMEM((_BPT, 16), jnp.float32),
            pltpu.VMEM((_BPT, 16), jnp.float32),
            pltpu.VMEM((_BPT,), jnp.float32),
            pltpu.SemaphoreType.DMA,
        ],
    )(_bg_body)


@functools.cache
def _sc_calls():
    mesh = plsc.VectorSubcoreMesh(core_axis_name="c", subcore_axis_name="s",
                                  num_cores=NC, num_subcores=NS)
    return (_mk_inv(mesh), _mk_supp(mesh), _mk_gather(mesh), _mk_edge(mesh),
            _mk_bg(mesh))


_TB = 128


def _t1_body(neigh_ref, hsum_ref, hl_ref, wqT_ref, wk_ref, wkT_ref, wvT_ref,
             out_ref):
    n3 = neigh_ref[0]
    u = hsum_ref[...] / hl_ref[...]
    q = jnp.dot(u, wqT_ref[0], preferred_element_type=jnp.float32)
    qk = jnp.dot(q, wk_ref[0], preferred_element_type=jnp.float32)
    sc = jnp.sum(n3 * qk[:, None, :], axis=2)
    sc = sc - jnp.max(sc, axis=1, keepdims=True)
    e = jnp.exp(sc)
    a = e / jnp.sum(e, axis=1, keepdims=True)
    nbar = jnp.sum(n3 * a[:, :, None], axis=1)
    ctx = jnp.dot(nbar, wkT_ref[0], preferred_element_type=jnp.float32)
    out_ref[0] = jnp.dot(ctx, wvT_ref[0], preferred_element_type=jnp.float32)


def _t1(neigh4, hsum, hl2, wqT, wk, wkT, wvT):
    grid = (HEADS, B // _TB)
    wspec = pl.BlockSpec((1, EMB, EMB), lambda h, b: (h, 0, 0))
    return pl.pallas_call(
        _t1_body,
        grid=grid,
        in_specs=[
            pl.BlockSpec((1, _TB, SAMPLE, EMB), lambda h, b: (h, b, 0, 0)),
            pl.BlockSpec((_TB, EMB), lambda h, b: (b, 0)),
            pl.BlockSpec((_TB, 1), lambda h, b: (b, 0)),
            wspec, wspec, wspec, wspec,
        ],
        out_specs=pl.BlockSpec((1, _TB, EMB), lambda h, b: (h, b, 0)),
        out_shape=jax.ShapeDtypeStruct((HEADS, B, EMB), jnp.float32),
    )(neigh4, hsum, hl2, wqT, wk, wkT, wvT)


_TB2 = 512


def _t2_body(hout_ref, irow_ref, gi_ref, gu_ref, du_ref, di_ref, bu_ref, bi_ref,
             woT_ref, guT_ref, gub_ref, giT_ref, gib_ref,
             l1T_ref, l1b_ref, l2T_ref, l2b_ref, l3w_ref, l3b_ref, out_ref):
    ue = jnp.zeros((_TB2, EMB), jnp.float32)
    for h in range(HEADS):
        ue = ue + jnp.dot(hout_ref[h], woT_ref[h * EMB:(h + 1) * EMB, :],
                          preferred_element_type=jnp.float32)
    ie = irow_ref[...]
    item_din = du_ref[:, 0:1] + 1.0
    user_din = di_ref[:, 0:1] + 1.0
    gih = gi_ref[...] / item_din
    guh = gu_ref[...] / user_din
    go_u = jnp.maximum(
        jnp.dot(guh, guT_ref[...], preferred_element_type=jnp.float32)
        + gub_ref[...], 0.0)
    go_i = jnp.maximum(
        jnp.dot(gih, giT_ref[...], preferred_element_type=jnp.float32)
        + gib_ref[...], 0.0)
    xx = jnp.concatenate([ue * ie, ue * go_i, go_u * ie, go_u * go_i], axis=1)
    x1 = jnp.tanh(jnp.dot(xx, l1T_ref[...], preferred_element_type=jnp.float32)
                  + l1b_ref[...])
    x2 = jnp.tanh(jnp.dot(x1, l2T_ref[...], preferred_element_type=jnp.float32)
                  + l2b_ref[...])
    x3 = jnp.sum(x2 * l3w_ref[...], axis=1, keepdims=True) + l3b_ref[...]
    out_ref[...] = x3 + bu_ref[...] + bi_ref[...]


def _t2(hout, irows, gi, gu, du, di, bu2, bi2,
        woT, guT, gub2, giT, gib2, l1T, l1b2, l2T, l2b2, l3w2, l3b2):
    grid = (B // _TB2,)
    full = lambda *s: pl.BlockSpec(s, lambda i: tuple(0 for _ in s))
    return pl.pallas_call(
        _t2_body,
        grid=grid,
        in_specs=[
            pl.BlockSpec((HEADS, _TB2, EMB), lambda i: (0, i, 0)),
            pl.BlockSpec((_TB2, EMB), lambda i: (i, 0)),
            pl.BlockSpec((_TB2, EMB), lambda i: (i, 0)),
            pl.BlockSpec((_TB2, EMB), lambda i: (i, 0)),
            pl.BlockSpec((_TB2, 16), lambda i: (i, 0)),
            pl.BlockSpec((_TB2, 16), lambda i: (i, 0)),
            pl.BlockSpec((_TB2, 1), lambda i: (i, 0)),
            pl.BlockSpec((_TB2, 1), lambda i: (i, 0)),
            full(HEADS * EMB, EMB),
            full(EMB, EMB),
            full(1, EMB),
            full(EMB, EMB),
            full(1, EMB),
            full(4 * EMB, 128),
            full(1, 128),
            full(128, EMB),
            full(1, EMB),
            full(1, EMB),
            full(1, 1),
        ],
        out_specs=pl.BlockSpec((_TB2, 1), lambda i: (i, 0)),
        out_shape=jax.ShapeDtypeStruct((B, 1), jnp.float32),
    )(hout, irows, gi, gu, du, di, bu2, bi2,
      woT, guT, gub2, giT, gib2, l1T, l1b2, l2T, l2b2, l3w2, l3b2)


def kernel(x, history, history_len, supp_users, edge_sparse, user_embedding,
           item_embedding, wq, wk, wv, w_out_W, gcn_user_W, gcn_user_b,
           gcn_item_W, gcn_item_b, l1_W, l1_b, l2_W, l2_b, l3_W, l3_b,
           user_bias, item_bias):
    uid = x[:, 0]
    iid = x[:, 1]
    histp = jnp.pad(history, ((0, 0), (0, HIST_P - HIST)))
    suppp = jnp.pad(supp_users, (0, SUPP_P - SUPP))
    skey = jax.random.key(42)
    sis = [jax.random.randint(jax.random.fold_in(skey, i), (B, SAMPLE), 0, SUPP)
           for i in range(HEADS)]
    si_flat = jnp.stack(sis).reshape(-1)
    srcp = jnp.pad(edge_sparse[0], (0, EPAD - E), constant_values=INV_N - 1)
    dstp = jnp.pad(edge_sparse[1], (0, EPAD - E), constant_values=INV_N - 1)
    ub = user_bias.reshape(-1)
    ib = item_bias.reshape(-1)

    _inv_call, _supp_call, _gather_call, _edge_call, _bg_call = _sc_calls()
    invu, invi = _inv_call(uid, iid)
    esupp = _supp_call(suppp, user_embedding)
    neigh, hsum, irows = _gather_call(si_flat, histp, iid, esupp, item_embedding)
    accu, acci, cntu, cnti = _edge_call(srcp, dstp, invu, invi,
                                        item_embedding, user_embedding)
    gi, gu, du, di, bu, bi = _bg_call(uid, iid, invu, invi,
                                      accu[0], accu[1], acci[0], acci[1],
                                      cntu[0], cntu[1], cnti[0], cnti[1],
                                      ub, ib)

    hl2 = history_len.astype(jnp.float32).reshape(B, 1)
    hout = _t1(neigh.reshape(HEADS, B, SAMPLE, EMB), hsum, hl2,
               wq.transpose(0, 2, 1), wk, wk.transpose(0, 2, 1),
               wv.transpose(0, 2, 1))
    out2 = _t2(hout, irows, gi, gu, du, di, bu.reshape(B, 1), bi.reshape(B, 1),
               w_out_W.T, gcn_user_W.T, gcn_user_b.reshape(1, EMB),
               gcn_item_W.T, gcn_item_b.reshape(1, EMB),
               l1_W.T, l1_b.reshape(1, -1), l2_W.T, l2_b.reshape(1, -1),
               l3_W.reshape(1, -1), l3_b.reshape(1, 1))
    return out2.reshape(-1)


# speedup vs baseline: 9.3866x; 1.0287x over previous
import functools

import jax
import jax.numpy as jnp
from jax import lax
from jax.experimental import pallas as pl
from jax.experimental.pallas import tpu as pltpu
from jax.experimental.pallas import tpu_sc as plsc

B = 4096
EMB = 64
HEADS = 4
SAMPLE = 100
HIST = 50
SUPP = 10000
E = 1000000
NU = 100000
NI = 100000
NC = 2
NS = 16
NW = NC * NS

INV_N = 100352
INV_PT = INV_N // NS
HIST_P = 56
SUPP_P = 10240
NEIGH_N = HEADS * B * SAMPLE
EPAD = 1024000
ECHUNK = 2000
CHUNKS_PER_TILE = EPAD // ECHUNK // NW
Q = 512
PEND = 2560
ACC_R = 4160
ACC_PT = ACC_R // NS
TRASH = 4096

def _wid():
    return lax.axis_index("s") * NC + lax.axis_index("c")


def _inv_body(uid_hbm, iid_hbm, invu_hbm, invi_hbm, neg_v, idx_v, val_v, sem):
    cid = lax.axis_index("c")
    sid = lax.axis_index("s")

    @pl.when(cid == 0)
    def _():
        def fill(i, _):
            neg_v[pl.ds(i * 16, 16)] = jnp.full((16,), -1, jnp.int32)
            return 0
        lax.fori_loop(0, INV_PT // 16, fill, 0)
        base = sid * INV_PT
        pltpu.sync_copy(neg_v, invu_hbm.at[pl.ds(base, INV_PT)])
        pltpu.sync_copy(neg_v, invi_hbm.at[pl.ds(base, INV_PT)])
        plsc.subcore_barrier()
        b0 = sid * (B // NS)
        lane = lax.iota(jnp.int32, 16)

        def fillv(i, _):
            val_v[pl.ds(i * 16, 16)] = b0 + i * 16 + lane
            return 0
        lax.fori_loop(0, (B // NS) // 16, fillv, 0)
        pltpu.sync_copy(uid_hbm.at[pl.ds(b0, B // NS)], idx_v)
        pltpu.async_copy(val_v, invu_hbm.at[idx_v], sem).wait()
        pltpu.sync_copy(iid_hbm.at[pl.ds(b0, B // NS)], idx_v)
        pltpu.async_copy(val_v, invi_hbm.at[idx_v], sem).wait()


def _mk_inv(mesh):
    return functools.partial(
        pl.kernel,
        out_type=(jax.ShapeDtypeStruct((INV_N,), jnp.int32),
                  jax.ShapeDtypeStruct((INV_N,), jnp.int32)),
        mesh=mesh,
        compiler_params=pltpu.CompilerParams(use_tc_tiling_on_sc=False, needs_layout_passes=False),
        scratch_types=[
            pltpu.VMEM((INV_PT,), jnp.int32),
            pltpu.VMEM((B // NS,), jnp.int32),
            pltpu.VMEM((B // NS,), jnp.int32),
            pltpu.SemaphoreType.DMA,
        ],
    )(_inv_body)


def _supp_body(supp_hbm, uemb_hbm, out_hbm, idx_v, rows_v, sem):
    base = _wid() * (SUPP_P // NW)
    pltpu.sync_copy(supp_hbm.at[pl.ds(base, SUPP_P // NW)], idx_v)
    pltpu.async_copy(uemb_hbm.at[idx_v], rows_v, sem).wait()
    pltpu.sync_copy(rows_v, out_hbm.at[pl.ds(base, SUPP_P // NW), :])


def _mk_supp(mesh):
    return functools.partial(
        pl.kernel,
        out_type=jax.ShapeDtypeStruct((SUPP_P, EMB), jnp.float32),
        mesh=mesh,
        compiler_params=pltpu.CompilerParams(use_tc_tiling_on_sc=False, needs_layout_passes=False),
        scratch_types=[
            pltpu.VMEM((SUPP_P // NW,), jnp.int32),
            pltpu.VMEM((SUPP_P // NW, EMB), jnp.float32),
            pltpu.SemaphoreType.DMA,
        ],
    )(_supp_body)


_NPT = NEIGH_N // NW
_NCK = 1024
_HG = 8
_HROWS = _HG * HIST_P


def _gather_body(si_hbm, hist_hbm, iid_hbm, esupp_hbm, iemb_hbm,
                 neigh_hbm, hsum_hbm, irows_hbm,
                 idx_v, rows_v, iidx_v, irow_v, hidx_v, hrows_v, hsum_v, sem):
    w = _wid()
    ibase = w * (B // NW)

    pltpu.sync_copy(iid_hbm.at[pl.ds(ibase, B // NW)], iidx_v)
    pltpu.async_copy(iemb_hbm.at[iidx_v], irow_v, sem).wait()
    pltpu.sync_copy(irow_v, irows_hbm.at[pl.ds(ibase, B // NW), :])

    def hg(g, _):
        hoff = (ibase + g * _HG) * HIST_P
        pltpu.sync_copy(hist_hbm.at[pl.ds(hoff, _HROWS)], hidx_v)
        pltpu.async_copy(iemb_hbm.at[hidx_v], hrows_v, sem).wait()
        z = jnp.zeros((16,), jnp.float32)
        for bl in range(_HG):
            def srow(k, acc):
                r = bl * HIST_P + k
                return tuple(acc[t] + hrows_v[r, pl.ds(t * 16, 16)]
                             for t in range(4))
            acc = lax.fori_loop(0, HIST, srow, (z, z, z, z))
            for t in range(4):
                hsum_v[g * _HG + bl, pl.ds(t * 16, 16)] = acc[t]
        return 0
    lax.fori_loop(0, (B // NW) // _HG, hg, 0)
    pltpu.sync_copy(hsum_v, hsum_hbm.at[pl.ds(ibase, B // NW), :])

    nbase = w * _NPT

    def chunk(i, _):
        off = nbase + i * _NCK
        pltpu.sync_copy(si_hbm.at[pl.ds(off, _NCK)], idx_v)
        pltpu.async_copy(esupp_hbm.at[idx_v], rows_v, sem).wait()
        pltpu.sync_copy(rows_v, neigh_hbm.at[pl.ds(off, _NCK), :])
        return 0
    lax.fori_loop(0, _NPT // _NCK, chunk, 0)


def _mk_gather(mesh):
    return functools.partial(
        pl.kernel,
        out_type=(jax.ShapeDtypeStruct((NEIGH_N, EMB), jnp.float32),
                  jax.ShapeDtypeStruct((B, EMB), jnp.float32),
                  jax.ShapeDtypeStruct((B, EMB), jnp.float32)),
        mesh=mesh,
        compiler_params=pltpu.CompilerParams(use_tc_tiling_on_sc=False, needs_layout_passes=False),
        scratch_types=[
            pltpu.VMEM((_NCK,), jnp.int32),
            pltpu.VMEM((_NCK, EMB), jnp.float32),
            pltpu.VMEM((B // NW,), jnp.int32),
            pltpu.VMEM((B // NW, EMB), jnp.float32),
            pltpu.VMEM((_HROWS,), jnp.int32),
            pltpu.VMEM((_HROWS, EMB), jnp.float32),
            pltpu.VMEM((B // NW, EMB), jnp.float32),
            pltpu.SemaphoreType.DMA,
        ],
    )(_gather_body)


RING = 8
_QSH = 9
_RMSK = RING * Q - 1


def _flush_row(t2, d2, table_hbm, acc_s, cnt_s, rows_v, ones_v, sem, f):
    rf = (f >> _QSH) & (RING - 1)
    pltpu.async_copy(table_hbm.at[d2.at[rf]], rows_v, sem).wait()
    pltpu.sync_copy(rows_v, acc_s.at[t2.at[rf]], add=True)
    pltpu.sync_copy(ones_v, cnt_s.at[t2.at[rf]], add=True)


def _drain(t2, d2, table_hbm, acc_s, cnt_s, rows_v, ones_v, sem, pu, f):
    nf = (pu - f) >> _QSH

    def fl(j, _):
        _flush_row(t2, d2, table_hbm, acc_s, cnt_s, rows_v, ones_v, sem,
                   f + j * Q)
        return 0
    lax.fori_loop(0, nf, fl, 0)
    return f + nf * Q


def _pad_tail(t2, d2, pu, f):
    lane = lax.iota(jnp.int32, 16)

    def pj(j, _):
        idx = f + j * 16 + lane
        m = idx >= pu
        r = idx & _RMSK
        plsc.store_scatter(t2, [r >> _QSH, r & (Q - 1)],
                           jnp.full((16,), TRASH, jnp.int32), mask=m)
        plsc.store_scatter(d2, [r >> _QSH, r & (Q - 1)],
                           jnp.zeros((16,), jnp.int32), mask=m)
        return 0
    lax.fori_loop(0, Q // 16, pj, 0)


def _edge_body(src_hbm, dst_hbm, invu_hbm, invi_hbm, iemb_hbm, uemb_hbm,
               accu_hbm, acci_hbm, cntu_hbm, cnti_hbm,
               s_v, d_v, tu_v, ti_v,
               pu_t, pu_d, pi_t, pi_s,
               rows_v, ones_v, zbuf, zcnt,
               accu_s, acci_s, cntu_s, cnti_s, sem, sem2):
    cid = lax.axis_index("c")
    sid = lax.axis_index("s")
    w = sid * NC + cid
    lane = lax.iota(jnp.int32, 16)
    onerow = jnp.where(lane == 0, 1.0, 0.0).astype(jnp.float32)

    def f1(i, _):
        ones_v[i, pl.ds(0, 16)] = onerow
        return 0
    lax.fori_loop(0, Q, f1, 0)

    zv = jnp.zeros((16,), jnp.float32)

    def fz(i, _):
        for t in range(4):
            zbuf[i, pl.ds(t * 16, 16)] = zv
        zcnt[i, pl.ds(0, 16)] = zv
        return 0
    lax.fori_loop(0, ACC_PT, fz, 0)

    rb = sid * ACC_PT
    pltpu.sync_copy(zbuf, accu_s.at[pl.ds(rb, ACC_PT), :])
    pltpu.sync_copy(zbuf, acci_s.at[pl.ds(rb, ACC_PT), :])
    pltpu.sync_copy(zcnt, cntu_s.at[pl.ds(rb, ACC_PT), :])
    pltpu.sync_copy(zcnt, cnti_s.at[pl.ds(rb, ACC_PT), :])
    plsc.subcore_barrier()

    def chunk_fn(c, carry):
        pu, fu, pi, fi = carry
        off = (w * CHUNKS_PER_TILE + c) * ECHUNK
        cs = pltpu.async_copy(src_hbm.at[pl.ds(off, ECHUNK)], s_v, sem)
        cd = pltpu.async_copy(dst_hbm.at[pl.ds(off, ECHUNK)], d_v, sem2)
        cs.wait()
        cd.wait()
        gu_ = pltpu.async_copy(invu_hbm.at[s_v], tu_v, sem)
        gi_ = pltpu.async_copy(invi_hbm.at[d_v], ti_v, sem2)
        gu_.wait()
        gi_.wait()

        def grp(g, pp):
            pu, pi = pp
            s16 = s_v[pl.ds(g * 16, 16)]
            d16 = d_v[pl.ds(g * 16, 16)]
            tu16 = tu_v[pl.ds(g * 16, 16)]
            ti16 = ti_v[pl.ds(g * 16, 16)]
            mu = tu16 >= 0
            mi = ti16 >= 0
            cu = plsc.cumsum(mu.astype(jnp.int32))
            iu = (pu + cu - 1) & _RMSK
            plsc.store_scatter(pu_t, [iu >> _QSH, iu & (Q - 1)], tu16, mask=mu)
            plsc.store_scatter(pu_d, [iu >> _QSH, iu & (Q - 1)], d16, mask=mu)
            pu = pu + jnp.sum(mu.astype(jnp.int32))
            ci = plsc.cumsum(mi.astype(jnp.int32))
            ii = (pi + ci - 1) & _RMSK
            plsc.store_scatter(pi_t, [ii >> _QSH, ii & (Q - 1)], ti16, mask=mi)
            plsc.store_scatter(pi_s, [ii >> _QSH, ii & (Q - 1)], s16, mask=mi)
            pi = pi + jnp.sum(mi.astype(jnp.int32))
            return (pu, pi)
        pu, pi = lax.fori_loop(0, ECHUNK // 16, grp, (pu, pi))
        fu = _drain(pu_t, pu_d, iemb_hbm, accu_s, cntu_s, rows_v, ones_v,
                    sem, pu, fu)
        fi = _drain(pi_t, pi_s, uemb_hbm, acci_s, cnti_s, rows_v, ones_v,
                    sem, pi, fi)
        return (pu, fu, pi, fi)

    z32 = jnp.int32(0)
    pu, fu, pi, fi = lax.fori_loop(0, CHUNKS_PER_TILE, chunk_fn,
                                   (z32, z32, z32, z32))

    _pad_tail(pu_t, pu_d, pu, fu)
    _flush_row(pu_t, pu_d, iemb_hbm, accu_s, cntu_s, rows_v, ones_v, sem, fu)
    _pad_tail(pi_t, pi_s, pi, fi)
    _flush_row(pi_t, pi_s, uemb_hbm, acci_s, cnti_s, rows_v, ones_v, sem, fi)

    plsc.subcore_barrier()
    pltpu.sync_copy(accu_s.at[pl.ds(rb, ACC_PT), :],
                    accu_hbm.at[cid, pl.ds(rb, ACC_PT), :])
    pltpu.sync_copy(acci_s.at[pl.ds(rb, ACC_PT), :],
                    acci_hbm.at[cid, pl.ds(rb, ACC_PT), :])
    pltpu.sync_copy(cntu_s.at[pl.ds(rb, ACC_PT), :],
                    cntu_hbm.at[cid, pl.ds(rb, ACC_PT), :])
    pltpu.sync_copy(cnti_s.at[pl.ds(rb, ACC_PT), :],
                    cnti_hbm.at[cid, pl.ds(rb, ACC_PT), :])


def _mk_edge(mesh):
    return functools.partial(
        pl.kernel,
        out_type=(jax.ShapeDtypeStruct((NC, ACC_R, EMB), jnp.float32),
                  jax.ShapeDtypeStruct((NC, ACC_R, EMB), jnp.float32),
                  jax.ShapeDtypeStruct((NC, ACC_R, 16), jnp.float32),
                  jax.ShapeDtypeStruct((NC, ACC_R, 16), jnp.float32)),
        mesh=mesh,
        compiler_params=pltpu.CompilerParams(use_tc_tiling_on_sc=False, needs_layout_passes=False),
        scratch_types=[
            pltpu.VMEM((ECHUNK,), jnp.int32),
            pltpu.VMEM((ECHUNK,), jnp.int32),
            pltpu.VMEM((ECHUNK,), jnp.int32),
            pltpu.VMEM((ECHUNK,), jnp.int32),
            pltpu.VMEM((RING, Q), jnp.int32),
            pltpu.VMEM((RING, Q), jnp.int32),
            pltpu.VMEM((RING, Q), jnp.int32),
            pltpu.VMEM((RING, Q), jnp.int32),
            pltpu.VMEM((Q, EMB), jnp.float32),
            pltpu.VMEM((Q, 16), jnp.float32),
            pltpu.VMEM((ACC_PT, EMB), jnp.float32),
            pltpu.VMEM((ACC_PT, 16), jnp.float32),
            pltpu.VMEM_SHARED((ACC_R, EMB), jnp.float32),
            pltpu.VMEM_SHARED((ACC_R, EMB), jnp.float32),
            pltpu.VMEM_SHARED((ACC_R, 16), jnp.float32),
            pltpu.VMEM_SHARED((ACC_R, 16), jnp.float32),
            pltpu.SemaphoreType.DMA,
            pltpu.SemaphoreType.DMA,
        ],
    )(_edge_body)


_BPT = B // NW


def _bg_body(uid_hbm, iid_hbm, invu_hbm, invi_hbm,
             au0, au1, ai0, ai1, cu0, cu1, ci0, ci1, ub_hbm, ib_hbm,
             gi_hbm, gu_hbm, du_hbm, di_hbm, bu_hbm, bi_hbm,
             uidv, iidv, tbu, tbi, r0, r1, c0, c1, bv, sem):
    base = _wid() * _BPT
    pltpu.sync_copy(uid_hbm.at[pl.ds(base, _BPT)], uidv)
    pltpu.sync_copy(iid_hbm.at[pl.ds(base, _BPT)], iidv)
    pltpu.async_copy(invu_hbm.at[uidv], tbu, sem).wait()
    pltpu.async_copy(invi_hbm.at[iidv], tbi, sem).wait()

    def addrows(j, _):
        for t in range(4):
            a = r0[j, pl.ds(t * 16, 16)]
            b = r1[j, pl.ds(t * 16, 16)]
            r0[j, pl.ds(t * 16, 16)] = a + b
        return 0

    def addcnt(j, _):
        a = c0[j, pl.ds(0, 16)]
        b = c1[j, pl.ds(0, 16)]
        c0[j, pl.ds(0, 16)] = a + b
        return 0

    pltpu.async_copy(au0.at[tbu], r0, sem).wait()
    pltpu.async_copy(au1.at[tbu], r1, sem).wait()
    lax.fori_loop(0, _BPT, addrows, 0)
    pltpu.sync_copy(r0, gi_hbm.at[pl.ds(base, _BPT), :])

    pltpu.async_copy(ai0.at[tbi], r0, sem).wait()
    pltpu.async_copy(ai1.at[tbi], r1, sem).wait()
    lax.fori_loop(0, _BPT, addrows, 0)
    pltpu.sync_copy(r0, gu_hbm.at[pl.ds(base, _BPT), :])

    pltpu.async_copy(cu0.at[tbu], c0, sem).wait()
    pltpu.async_copy(cu1.at[tbu], c1, sem).wait()
    lax.fori_loop(0, _BPT, addcnt, 0)
    pltpu.sync_copy(c0, du_hbm.at[pl.ds(base, _BPT), :])

    pltpu.async_copy(ci0.at[tbi], c0, sem).wait()
    pltpu.async_copy(ci1.at[tbi], c1, sem).wait()
    lax.fori_loop(0, _BPT, addcnt, 0)
    pltpu.sync_copy(c0, di_hbm.at[pl.ds(base, _BPT), :])

    pltpu.async_copy(ub_hbm.at[uidv], bv, sem).wait()
    pltpu.sync_copy(bv, bu_hbm.at[pl.ds(base, _BPT)])
    pltpu.async_copy(ib_hbm.at[iidv], bv, sem).wait()
    pltpu.sync_copy(bv, bi_hbm.at[pl.ds(base, _BPT)])


def _mk_bg(mesh):
    return functools.partial(
        pl.kernel,
        out_type=(jax.ShapeDtypeStruct((B, EMB), jnp.float32),
                  jax.ShapeDtypeStruct((B, EMB), jnp.float32),
                  jax.ShapeDtypeStruct((B, 16), jnp.float32),
                  jax.ShapeDtypeStruct((B, 16), jnp.float32),
                  jax.ShapeDtypeStruct((B,), jnp.float32),
                  jax.ShapeDtypeStruct((B,), jnp.float32)),
        mesh=mesh,
        compiler_params=pltpu.CompilerParams(use_tc_tiling_on_sc=False, needs_layout_passes=False),
        scratch_types=[
            pltpu.VMEM((_BPT,), jnp.int32),
            pltpu.VMEM((_BPT,), jnp.int32),
            pltpu.VMEM((_BPT,), jnp.int32),
            pltpu.VMEM((_BPT,), jnp.int32),
            pltpu.VMEM((_BPT, EMB), jnp.float32),
            pltpu.VMEM((_BPT, EMB), jnp.float32),
            pltpu.VMEM((_BPT, 16), jnp.float32),
            pltpu.VMEM((_BPT, 16), jnp.float32),
            pltpu.VMEM((_BPT,), jnp.float32),
            pltpu.SemaphoreType.DMA,
        ],
    )(_bg_body)


@functools.cache
def _sc_calls():
    mesh = plsc.VectorSubcoreMesh(core_axis_name="c", subcore_axis_name="s",
                                  num_cores=NC, num_subcores=NS)
    return (_mk_inv(mesh), _mk_supp(mesh), _mk_gather(mesh), _mk_edge(mesh),
            _mk_bg(mesh))


_TB = 128


def _t1_body(neigh_ref, hsum_ref, hl_ref, wqT_ref, wk_ref, wkT_ref, wvT_ref,
             out_ref):
    n3 = neigh_ref[0]
    u = hsum_ref[...] / hl_ref[...]
    q = jnp.dot(u, wqT_ref[0], preferred_element_type=jnp.float32)
    qk = jnp.dot(q, wk_ref[0], preferred_element_type=jnp.float32,
                 precision=lax.Precision.HIGHEST)
    sc = jnp.sum(n3 * qk[:, None, :], axis=2)
    sc = sc - jnp.max(sc, axis=1, keepdims=True)
    e = jnp.exp(sc)
    a = e / jnp.sum(e, axis=1, keepdims=True)
    nbar = jnp.sum(n3 * a[:, :, None], axis=1)
    ctx = jnp.dot(nbar, wkT_ref[0], preferred_element_type=jnp.float32,
                  precision=lax.Precision.HIGHEST)
    out_ref[0] = jnp.dot(ctx, wvT_ref[0], preferred_element_type=jnp.float32)


def _t1(neigh4, hsum, hl2, wqT, wk, wkT, wvT):
    grid = (HEADS, B // _TB)
    wspec = pl.BlockSpec((1, EMB, EMB), lambda h, b: (h, 0, 0))
    return pl.pallas_call(
        _t1_body,
        grid=grid,
        in_specs=[
            pl.BlockSpec((1, _TB, SAMPLE, EMB), lambda h, b: (h, b, 0, 0)),
            pl.BlockSpec((_TB, EMB), lambda h, b: (b, 0)),
            pl.BlockSpec((_TB, 1), lambda h, b: (b, 0)),
            wspec, wspec, wspec, wspec,
        ],
        out_specs=pl.BlockSpec((1, _TB, EMB), lambda h, b: (h, b, 0)),
        out_shape=jax.ShapeDtypeStruct((HEADS, B, EMB), jnp.float32),
    )(neigh4, hsum, hl2, wqT, wk, wkT, wvT)


_TB2 = 512


def _t2_body(hout_ref, irow_ref, gi_ref, gu_ref, du_ref, di_ref, bu_ref, bi_ref,
             woT_ref, guT_ref, gub_ref, giT_ref, gib_ref,
             l1T_ref, l1b_ref, l2T_ref, l2b_ref, l3w_ref, l3b_ref, out_ref):
    ue = jnp.zeros((_TB2, EMB), jnp.float32)
    for h in range(HEADS):
        ue = ue + jnp.dot(hout_ref[h], woT_ref[h * EMB:(h + 1) * EMB, :],
                          preferred_element_type=jnp.float32)
    ie = irow_ref[...]
    item_din = du_ref[:, 0:1] + 1.0
    user_din = di_ref[:, 0:1] + 1.0
    gih = gi_ref[...] / item_din
    guh = gu_ref[...] / user_din
    go_u = jnp.maximum(
        jnp.dot(guh, guT_ref[...], preferred_element_type=jnp.float32)
        + gub_ref[...], 0.0)
    go_i = jnp.maximum(
        jnp.dot(gih, giT_ref[...], preferred_element_type=jnp.float32)
        + gib_ref[...], 0.0)
    xx = jnp.concatenate([ue * ie, ue * go_i, go_u * ie, go_u * go_i], axis=1)
    x1 = jnp.tanh(jnp.dot(xx, l1T_ref[...], preferred_element_type=jnp.float32)
                  + l1b_ref[...])
    x2 = jnp.tanh(jnp.dot(x1, l2T_ref[...], preferred_element_type=jnp.float32)
                  + l2b_ref[...])
    x3 = jnp.sum(x2 * l3w_ref[...], axis=1, keepdims=True) + l3b_ref[...]
    out_ref[...] = x3 + bu_ref[...] + bi_ref[...]


def _t2(hout, irows, gi, gu, du, di, bu2, bi2,
        woT, guT, gub2, giT, gib2, l1T, l1b2, l2T, l2b2, l3w2, l3b2):
    grid = (B // _TB2,)
    full = lambda *s: pl.BlockSpec(s, lambda i: tuple(0 for _ in s))
    return pl.pallas_call(
        _t2_body,
        grid=grid,
        in_specs=[
            pl.BlockSpec((HEADS, _TB2, EMB), lambda i: (0, i, 0)),
            pl.BlockSpec((_TB2, EMB), lambda i: (i, 0)),
            pl.BlockSpec((_TB2, EMB), lambda i: (i, 0)),
            pl.BlockSpec((_TB2, EMB), lambda i: (i, 0)),
            pl.BlockSpec((_TB2, 16), lambda i: (i, 0)),
            pl.BlockSpec((_TB2, 16), lambda i: (i, 0)),
            pl.BlockSpec((_TB2, 1), lambda i: (i, 0)),
            pl.BlockSpec((_TB2, 1), lambda i: (i, 0)),
            full(HEADS * EMB, EMB),
            full(EMB, EMB),
            full(1, EMB),
            full(EMB, EMB),
            full(1, EMB),
            full(4 * EMB, 128),
            full(1, 128),
            full(128, EMB),
            full(1, EMB),
            full(1, EMB),
            full(1, 1),
        ],
        out_specs=pl.BlockSpec((_TB2, 1), lambda i: (i, 0)),
        out_shape=jax.ShapeDtypeStruct((B, 1), jnp.float32),
    )(hout, irows, gi, gu, du, di, bu2, bi2,
      woT, guT, gub2, giT, gib2, l1T, l1b2, l2T, l2b2, l3w2, l3b2)


def kernel(x, history, history_len, supp_users, edge_sparse, user_embedding,
           item_embedding, wq, wk, wv, w_out_W, gcn_user_W, gcn_user_b,
           gcn_item_W, gcn_item_b, l1_W, l1_b, l2_W, l2_b, l3_W, l3_b,
           user_bias, item_bias):
    uid = x[:, 0]
    iid = x[:, 1]
    histp = jnp.pad(history, ((0, 0), (0, HIST_P - HIST)))
    suppp = jnp.pad(supp_users, (0, SUPP_P - SUPP))
    skey = jax.random.key(42)
    sis = [jax.random.randint(jax.random.fold_in(skey, i), (B, SAMPLE), 0, SUPP)
           for i in range(HEADS)]
    si_flat = jnp.stack(sis).reshape(-1)
    srcp = jnp.pad(edge_sparse[0], (0, EPAD - E), constant_values=INV_N - 1)
    dstp = jnp.pad(edge_sparse[1], (0, EPAD - E), constant_values=INV_N - 1)
    ub = user_bias.reshape(-1)
    ib = item_bias.reshape(-1)

    _inv_call, _supp_call, _gather_call, _edge_call, _bg_call = _sc_calls()
    invu, invi = _inv_call(uid, iid)
    esupp = _supp_call(suppp, user_embedding)
    neigh, hsum, irows = _gather_call(si_flat, histp.reshape(-1), iid, esupp,
                                      item_embedding)
    accu, acci, cntu, cnti = _edge_call(srcp, dstp, invu, invi,
                                        item_embedding, user_embedding)
    gi, gu, du, di, bu, bi = _bg_call(uid, iid, invu, invi,
                                      accu[0], accu[1], acci[0], acci[1],
                                      cntu[0], cntu[1], cnti[0], cnti[1],
                                      ub, ib)

    hl2 = history_len.astype(jnp.float32).reshape(B, 1)
    hout = _t1(neigh.reshape(HEADS, B, SAMPLE, EMB), hsum, hl2,
               wq.transpose(0, 2, 1), wk, wk.transpose(0, 2, 1),
               wv.transpose(0, 2, 1))
    out2 = _t2(hout, irows, gi, gu, du, di, bu.reshape(B, 1), bi.reshape(B, 1),
               w_out_W.T, gcn_user_W.T, gcn_user_b.reshape(1, EMB),
               gcn_item_W.T, gcn_item_b.reshape(1, EMB),
               l1_W.T, l1_b.reshape(1, -1), l2_W.T, l2_b.reshape(1, -1),
               l3_W.reshape(1, -1), l3_b.reshape(1, 1))
    return out2.reshape(-1)


# speedup vs baseline: 9.4175x; 1.0033x over previous
import functools

import jax
import jax.numpy as jnp
from jax import lax
from jax.experimental import pallas as pl
from jax.experimental.pallas import tpu as pltpu
from jax.experimental.pallas import tpu_sc as plsc

B = 4096
EMB = 64
HEADS = 4
SAMPLE = 100
HIST = 50
SUPP = 10000
E = 1000000
NU = 100000
NI = 100000
NC = 2
NS = 16
NW = NC * NS

INV_N = 100352
INV_PT = INV_N // NS
HIST_P = 56
SUPP_P = 10240
NEIGH_N = HEADS * B * SAMPLE
EPAD = 1024000
ECHUNK = 2000
CHUNKS_PER_TILE = EPAD // ECHUNK // NW
Q = 512
PEND = 2560
ACC_R = 4160
ACC_PT = ACC_R // NS
TRASH = 4096

def _wid():
    return lax.axis_index("s") * NC + lax.axis_index("c")


def _inv_body(uid_hbm, iid_hbm, invu_hbm, invi_hbm, neg_v, idx_v, val_v, sem):
    cid = lax.axis_index("c")
    sid = lax.axis_index("s")

    @pl.when(cid == 0)
    def _():
        def fill(i, _):
            neg_v[pl.ds(i * 16, 16)] = jnp.full((16,), -1, jnp.int32)
            return 0
        lax.fori_loop(0, INV_PT // 16, fill, 0)
        base = sid * INV_PT
        pltpu.sync_copy(neg_v, invu_hbm.at[pl.ds(base, INV_PT)])
        pltpu.sync_copy(neg_v, invi_hbm.at[pl.ds(base, INV_PT)])
        plsc.subcore_barrier()
        b0 = sid * (B // NS)
        lane = lax.iota(jnp.int32, 16)

        def fillv(i, _):
            val_v[pl.ds(i * 16, 16)] = b0 + i * 16 + lane
            return 0
        lax.fori_loop(0, (B // NS) // 16, fillv, 0)
        pltpu.sync_copy(uid_hbm.at[pl.ds(b0, B // NS)], idx_v)
        pltpu.async_copy(val_v, invu_hbm.at[idx_v], sem).wait()
        pltpu.sync_copy(iid_hbm.at[pl.ds(b0, B // NS)], idx_v)
        pltpu.async_copy(val_v, invi_hbm.at[idx_v], sem).wait()


def _mk_inv(mesh):
    return functools.partial(
        pl.kernel,
        out_type=(jax.ShapeDtypeStruct((INV_N,), jnp.int32),
                  jax.ShapeDtypeStruct((INV_N,), jnp.int32)),
        mesh=mesh,
        compiler_params=pltpu.CompilerParams(use_tc_tiling_on_sc=False, needs_layout_passes=False),
        scratch_types=[
            pltpu.VMEM((INV_PT,), jnp.int32),
            pltpu.VMEM((B // NS,), jnp.int32),
            pltpu.VMEM((B // NS,), jnp.int32),
            pltpu.SemaphoreType.DMA,
        ],
    )(_inv_body)


def _supp_body(supp_hbm, uemb_hbm, out_hbm, idx_v, rows_v, sem):
    base = _wid() * (SUPP_P // NW)
    pltpu.sync_copy(supp_hbm.at[pl.ds(base, SUPP_P // NW)], idx_v)
    pltpu.async_copy(uemb_hbm.at[idx_v], rows_v, sem).wait()
    pltpu.sync_copy(rows_v, out_hbm.at[pl.ds(base, SUPP_P // NW), :])


def _mk_supp(mesh):
    return functools.partial(
        pl.kernel,
        out_type=jax.ShapeDtypeStruct((SUPP_P, EMB), jnp.float32),
        mesh=mesh,
        compiler_params=pltpu.CompilerParams(use_tc_tiling_on_sc=False, needs_layout_passes=False),
        scratch_types=[
            pltpu.VMEM((SUPP_P // NW,), jnp.int32),
            pltpu.VMEM((SUPP_P // NW, EMB), jnp.float32),
            pltpu.SemaphoreType.DMA,
        ],
    )(_supp_body)


_NPT = NEIGH_N // NW
_NCK = 640
_HG = 4
_HROWS = _HG * HIST_P


def _gather_body(si_hbm, hist_hbm, iid_hbm, esupp_hbm, iemb_hbm,
                 neigh_hbm, hsum_hbm, irows_hbm,
                 idx_v, rows_v, idx2_v, rows2_v, iidx_v, irow_v, hidx_v,
                 hrows_v, hsum_v, sem, sem2, wsem, wsem2):
    w = _wid()
    ibase = w * (B // NW)

    pltpu.sync_copy(iid_hbm.at[pl.ds(ibase, B // NW)], iidx_v)
    pltpu.async_copy(iemb_hbm.at[iidx_v], irow_v, sem).wait()
    pltpu.sync_copy(irow_v, irows_hbm.at[pl.ds(ibase, B // NW), :])

    def hg(g, _):
        hoff = (ibase + g * _HG) * HIST_P
        pltpu.sync_copy(hist_hbm.at[pl.ds(hoff, _HROWS)], hidx_v)
        pltpu.async_copy(iemb_hbm.at[hidx_v], hrows_v, sem).wait()
        z = jnp.zeros((16,), jnp.float32)
        for bl in range(_HG):
            def srow(k, acc):
                r = bl * HIST_P + k
                return tuple(acc[t] + hrows_v[r, pl.ds(t * 16, 16)]
                             for t in range(4))
            acc = lax.fori_loop(0, HIST, srow, (z, z, z, z))
            for t in range(4):
                hsum_v[g * _HG + bl, pl.ds(t * 16, 16)] = acc[t]
        return 0
    lax.fori_loop(0, (B // NW) // _HG, hg, 0)
    pltpu.sync_copy(hsum_v, hsum_hbm.at[pl.ds(ibase, B // NW), :])

    nbase = w * _NPT

    def pair(k, _):
        o0 = nbase + (2 * k) * _NCK
        o1 = o0 + _NCK
        pltpu.sync_copy(si_hbm.at[pl.ds(o0, _NCK)], idx_v)
        pltpu.sync_copy(si_hbm.at[pl.ds(o1, _NCK)], idx2_v)
        g0 = pltpu.async_copy(esupp_hbm.at[idx_v], rows_v, sem)
        g1 = pltpu.async_copy(esupp_hbm.at[idx2_v], rows2_v, sem2)
        g0.wait()
        w0 = pltpu.async_copy(rows_v, neigh_hbm.at[pl.ds(o0, _NCK), :], wsem)
        g1.wait()
        w1 = pltpu.async_copy(rows2_v, neigh_hbm.at[pl.ds(o1, _NCK), :], wsem2)
        w0.wait()
        w1.wait()
        return 0
    lax.fori_loop(0, _NPT // _NCK // 2, pair, 0)


def _mk_gather(mesh):
    return functools.partial(
        pl.kernel,
        out_type=(jax.ShapeDtypeStruct((NEIGH_N, EMB), jnp.float32),
                  jax.ShapeDtypeStruct((B, EMB), jnp.float32),
                  jax.ShapeDtypeStruct((B, EMB), jnp.float32)),
        mesh=mesh,
        compiler_params=pltpu.CompilerParams(use_tc_tiling_on_sc=False, needs_layout_passes=False),
        scratch_types=[
            pltpu.VMEM((_NCK,), jnp.int32),
            pltpu.VMEM((_NCK, EMB), jnp.float32),
            pltpu.VMEM((_NCK,), jnp.int32),
            pltpu.VMEM((_NCK, EMB), jnp.float32),
            pltpu.VMEM((B // NW,), jnp.int32),
            pltpu.VMEM((B // NW, EMB), jnp.float32),
            pltpu.VMEM((_HROWS,), jnp.int32),
            pltpu.VMEM((_HROWS, EMB), jnp.float32),
            pltpu.VMEM((B // NW, EMB), jnp.float32),
            pltpu.SemaphoreType.DMA,
            pltpu.SemaphoreType.DMA,
            pltpu.SemaphoreType.DMA,
            pltpu.SemaphoreType.DMA,
        ],
    )(_gather_body)


RING = 8
_QSH = 9
_RMSK = RING * Q - 1


def _flush_row(t2, d2, table_hbm, acc_s, cnt_s, rows_v, ones_v, sem, f):
    rf = (f >> _QSH) & (RING - 1)
    pltpu.async_copy(table_hbm.at[d2.at[rf]], rows_v, sem).wait()
    pltpu.sync_copy(rows_v, acc_s.at[t2.at[rf]], add=True)
    pltpu.sync_copy(ones_v, cnt_s.at[t2.at[rf]], add=True)


def _drain(t2, d2, table_hbm, acc_s, cnt_s, rows_v, ones_v, sem, pu, f):
    nf = (pu - f) >> _QSH

    def fl(j, _):
        _flush_row(t2, d2, table_hbm, acc_s, cnt_s, rows_v, ones_v, sem,
                   f + j * Q)
        return 0
    lax.fori_loop(0, nf, fl, 0)
    return f + nf * Q


def _pad_tail(t2, d2, pu, f):
    lane = lax.iota(jnp.int32, 16)

    def pj(j, _):
        idx = f + j * 16 + lane
        m = idx >= pu
        r = idx & _RMSK
        plsc.store_scatter(t2, [r >> _QSH, r & (Q - 1)],
                           jnp.full((16,), TRASH, jnp.int32), mask=m)
        plsc.store_scatter(d2, [r >> _QSH, r & (Q - 1)],
                           jnp.zeros((16,), jnp.int32), mask=m)
        return 0
    lax.fori_loop(0, Q // 16, pj, 0)


def _edge_body(src_hbm, dst_hbm, invu_hbm, invi_hbm, iemb_hbm, uemb_hbm,
               accu_hbm, acci_hbm, cntu_hbm, cnti_hbm,
               s_v, d_v, tu_v, ti_v,
               pu_t, pu_d, pi_t, pi_s,
               rows_v, ones_v, zbuf, zcnt,
               accu_s, acci_s, cntu_s, cnti_s, sem, sem2):
    cid = lax.axis_index("c")
    sid = lax.axis_index("s")
    w = sid * NC + cid
    lane = lax.iota(jnp.int32, 16)
    onerow = jnp.where(lane == 0, 1.0, 0.0).astype(jnp.float32)

    def f1(i, _):
        ones_v[i, pl.ds(0, 16)] = onerow
        return 0
    lax.fori_loop(0, Q, f1, 0)

    zv = jnp.zeros((16,), jnp.float32)

    def fz(i, _):
        for t in range(4):
            zbuf[i, pl.ds(t * 16, 16)] = zv
        zcnt[i, pl.ds(0, 16)] = zv
        return 0
    lax.fori_loop(0, ACC_PT, fz, 0)

    rb = sid * ACC_PT
    pltpu.sync_copy(zbuf, accu_s.at[pl.ds(rb, ACC_PT), :])
    pltpu.sync_copy(zbuf, acci_s.at[pl.ds(rb, ACC_PT), :])
    pltpu.sync_copy(zcnt, cntu_s.at[pl.ds(rb, ACC_PT), :])
    pltpu.sync_copy(zcnt, cnti_s.at[pl.ds(rb, ACC_PT), :])
    plsc.subcore_barrier()

    def chunk_fn(c, carry):
        pu, fu, pi, fi = carry
        off = (w * CHUNKS_PER_TILE + c) * ECHUNK
        cs = pltpu.async_copy(src_hbm.at[pl.ds(off, ECHUNK)], s_v, sem)
        cd = pltpu.async_copy(dst_hbm.at[pl.ds(off, ECHUNK)], d_v, sem2)
        cs.wait()
        cd.wait()
        gu_ = pltpu.async_copy(invu_hbm.at[s_v], tu_v, sem)
        gi_ = pltpu.async_copy(invi_hbm.at[d_v], ti_v, sem2)
        gu_.wait()
        gi_.wait()

        def grp(g, pp):
            pu, pi = pp
            s16 = s_v[pl.ds(g * 16, 16)]
            d16 = d_v[pl.ds(g * 16, 16)]
            tu16 = tu_v[pl.ds(g * 16, 16)]
            ti16 = ti_v[pl.ds(g * 16, 16)]
            mu = tu16 >= 0
            mi = ti16 >= 0
            cu = plsc.cumsum(mu.astype(jnp.int32))
            iu = (pu + cu - 1) & _RMSK
            plsc.store_scatter(pu_t, [iu >> _QSH, iu & (Q - 1)], tu16, mask=mu)
            plsc.store_scatter(pu_d, [iu >> _QSH, iu & (Q - 1)], d16, mask=mu)
            pu = pu + jnp.sum(mu.astype(jnp.int32))
            ci = plsc.cumsum(mi.astype(jnp.int32))
            ii = (pi + ci - 1) & _RMSK
            plsc.store_scatter(pi_t, [ii >> _QSH, ii & (Q - 1)], ti16, mask=mi)
            plsc.store_scatter(pi_s, [ii >> _QSH, ii & (Q - 1)], s16, mask=mi)
            pi = pi + jnp.sum(mi.astype(jnp.int32))
            return (pu, pi)
        pu, pi = lax.fori_loop(0, ECHUNK // 16, grp, (pu, pi))
        fu = _drain(pu_t, pu_d, iemb_hbm, accu_s, cntu_s, rows_v, ones_v,
                    sem, pu, fu)
        fi = _drain(pi_t, pi_s, uemb_hbm, acci_s, cnti_s, rows_v, ones_v,
                    sem, pi, fi)
        return (pu, fu, pi, fi)

    z32 = jnp.int32(0)
    pu, fu, pi, fi = lax.fori_loop(0, CHUNKS_PER_TILE, chunk_fn,
                                   (z32, z32, z32, z32))

    _pad_tail(pu_t, pu_d, pu, fu)
    _flush_row(pu_t, pu_d, iemb_hbm, accu_s, cntu_s, rows_v, ones_v, sem, fu)
    _pad_tail(pi_t, pi_s, pi, fi)
    _flush_row(pi_t, pi_s, uemb_hbm, acci_s, cnti_s, rows_v, ones_v, sem, fi)

    plsc.subcore_barrier()
    pltpu.sync_copy(accu_s.at[pl.ds(rb, ACC_PT), :],
                    accu_hbm.at[cid, pl.ds(rb, ACC_PT), :])
    pltpu.sync_copy(acci_s.at[pl.ds(rb, ACC_PT), :],
                    acci_hbm.at[cid, pl.ds(rb, ACC_PT), :])
    pltpu.sync_copy(cntu_s.at[pl.ds(rb, ACC_PT), :],
                    cntu_hbm.at[cid, pl.ds(rb, ACC_PT), :])
    pltpu.sync_copy(cnti_s.at[pl.ds(rb, ACC_PT), :],
                    cnti_hbm.at[cid, pl.ds(rb, ACC_PT), :])


def _mk_edge(mesh):
    return functools.partial(
        pl.kernel,
        out_type=(jax.ShapeDtypeStruct((NC, ACC_R, EMB), jnp.float32),
                  jax.ShapeDtypeStruct((NC, ACC_R, EMB), jnp.float32),
                  jax.ShapeDtypeStruct((NC, ACC_R, 16), jnp.float32),
                  jax.ShapeDtypeStruct((NC, ACC_R, 16), jnp.float32)),
        mesh=mesh,
        compiler_params=pltpu.CompilerParams(use_tc_tiling_on_sc=False, needs_layout_passes=False),
        scratch_types=[
            pltpu.VMEM((ECHUNK,), jnp.int32),
            pltpu.VMEM((ECHUNK,), jnp.int32),
            pltpu.VMEM((ECHUNK,), jnp.int32),
            pltpu.VMEM((ECHUNK,), jnp.int32),
            pltpu.VMEM((RING, Q), jnp.int32),
            pltpu.VMEM((RING, Q), jnp.int32),
            pltpu.VMEM((RING, Q), jnp.int32),
            pltpu.VMEM((RING, Q), jnp.int32),
            pltpu.VMEM((Q, EMB), jnp.float32),
            pltpu.VMEM((Q, 16), jnp.float32),
            pltpu.VMEM((ACC_PT, EMB), jnp.float32),
            pltpu.VMEM((ACC_PT, 16), jnp.float32),
            pltpu.VMEM_SHARED((ACC_R, EMB), jnp.float32),
            pltpu.VMEM_SHARED((ACC_R, EMB), jnp.float32),
            pltpu.VMEM_SHARED((ACC_R, 16), jnp.float32),
            pltpu.VMEM_SHARED((ACC_R, 16), jnp.float32),
            pltpu.SemaphoreType.DMA,
            pltpu.SemaphoreType.DMA,
        ],
    )(_edge_body)


_BPT = B // NW


def _bg_body(uid_hbm, iid_hbm, invu_hbm, invi_hbm,
             au0, au1, ai0, ai1, cu0, cu1, ci0, ci1, ub_hbm, ib_hbm,
             gi_hbm, gu_hbm, du_hbm, di_hbm, bu_hbm, bi_hbm,
             uidv, iidv, tbu, tbi, r0, r1, c0, c1, bv, sem):
    base = _wid() * _BPT
    pltpu.sync_copy(uid_hbm.at[pl.ds(base, _BPT)], uidv)
    pltpu.sync_copy(iid_hbm.at[pl.ds(base, _BPT)], iidv)
    pltpu.async_copy(invu_hbm.at[uidv], tbu, sem).wait()
    pltpu.async_copy(invi_hbm.at[iidv], tbi, sem).wait()

    def addrows(j, _):
        for t in range(4):
            a = r0[j, pl.ds(t * 16, 16)]
            b = r1[j, pl.ds(t * 16, 16)]
            r0[j, pl.ds(t * 16, 16)] = a + b
        return 0

    def addcnt(j, _):
        a = c0[j, pl.ds(0, 16)]
        b = c1[j, pl.ds(0, 16)]
        c0[j, pl.ds(0, 16)] = a + b
        return 0

    pltpu.async_copy(au0.at[tbu], r0, sem).wait()
    pltpu.async_copy(au1.at[tbu], r1, sem).wait()
    lax.fori_loop(0, _BPT, addrows, 0)
    pltpu.sync_copy(r0, gi_hbm.at[pl.ds(base, _BPT), :])

    pltpu.async_copy(ai0.at[tbi], r0, sem).wait()
    pltpu.async_copy(ai1.at[tbi], r1, sem).wait()
    lax.fori_loop(0, _BPT, addrows, 0)
    pltpu.sync_copy(r0, gu_hbm.at[pl.ds(base, _BPT), :])

    pltpu.async_copy(cu0.at[tbu], c0, sem).wait()
    pltpu.async_copy(cu1.at[tbu], c1, sem).wait()
    lax.fori_loop(0, _BPT, addcnt, 0)
    pltpu.sync_copy(c0, du_hbm.at[pl.ds(base, _BPT), :])

    pltpu.async_copy(ci0.at[tbi], c0, sem).wait()
    pltpu.async_copy(ci1.at[tbi], c1, sem).wait()
    lax.fori_loop(0, _BPT, addcnt, 0)
    pltpu.sync_copy(c0, di_hbm.at[pl.ds(base, _BPT), :])

    pltpu.async_copy(ub_hbm.at[uidv], bv, sem).wait()
    pltpu.sync_copy(bv, bu_hbm.at[pl.ds(base, _BPT)])
    pltpu.async_copy(ib_hbm.at[iidv], bv, sem).wait()
    pltpu.sync_copy(bv, bi_hbm.at[pl.ds(base, _BPT)])


def _mk_bg(mesh):
    return functools.partial(
        pl.kernel,
        out_type=(jax.ShapeDtypeStruct((B, EMB), jnp.float32),
                  jax.ShapeDtypeStruct((B, EMB), jnp.float32),
                  jax.ShapeDtypeStruct((B, 16), jnp.float32),
                  jax.ShapeDtypeStruct((B, 16), jnp.float32),
                  jax.ShapeDtypeStruct((B,), jnp.float32),
                  jax.ShapeDtypeStruct((B,), jnp.float32)),
        mesh=mesh,
        compiler_params=pltpu.CompilerParams(use_tc_tiling_on_sc=False, needs_layout_passes=False),
        scratch_types=[
            pltpu.VMEM((_BPT,), jnp.int32),
            pltpu.VMEM((_BPT,), jnp.int32),
            pltpu.VMEM((_BPT,), jnp.int32),
            pltpu.VMEM((_BPT,), jnp.int32),
            pltpu.VMEM((_BPT, EMB), jnp.float32),
            pltpu.VMEM((_BPT, EMB), jnp.float32),
            pltpu.VMEM((_BPT, 16), jnp.float32),
            pltpu.VMEM((_BPT, 16), jnp.float32),
            pltpu.VMEM((_BPT,), jnp.float32),
            pltpu.SemaphoreType.DMA,
        ],
    )(_bg_body)


@functools.cache
def _sc_calls():
    mesh = plsc.VectorSubcoreMesh(core_axis_name="c", subcore_axis_name="s",
                                  num_cores=NC, num_subcores=NS)
    return (_mk_inv(mesh), _mk_supp(mesh), _mk_gather(mesh), _mk_edge(mesh),
            _mk_bg(mesh))


_TB = 128


def _t1_body(neigh_ref, hsum_ref, hl_ref, wqT_ref, wk_ref, wkT_ref, wvT_ref,
             out_ref):
    n3 = neigh_ref[0]
    u = hsum_ref[...] / hl_ref[...]
    q = jnp.dot(u, wqT_ref[0], preferred_element_type=jnp.float32)
    qk = jnp.dot(q, wk_ref[0], preferred_element_type=jnp.float32,
                 precision=lax.Precision.HIGHEST)
    sc = jnp.sum(n3 * qk[:, None, :], axis=2)
    sc = sc - jnp.max(sc, axis=1, keepdims=True)
    e = jnp.exp(sc)
    a = e / jnp.sum(e, axis=1, keepdims=True)
    nbar = jnp.sum(n3 * a[:, :, None], axis=1)
    ctx = jnp.dot(nbar, wkT_ref[0], preferred_element_type=jnp.float32,
                  precision=lax.Precision.HIGHEST)
    out_ref[0] = jnp.dot(ctx, wvT_ref[0], preferred_element_type=jnp.float32)


def _t1(neigh4, hsum, hl2, wqT, wk, wkT, wvT):
    grid = (HEADS, B // _TB)
    wspec = pl.BlockSpec((1, EMB, EMB), lambda h, b: (h, 0, 0))
    return pl.pallas_call(
        _t1_body,
        grid=grid,
        in_specs=[
            pl.BlockSpec((1, _TB, SAMPLE, EMB), lambda h, b: (h, b, 0, 0)),
            pl.BlockSpec((_TB, EMB), lambda h, b: (b, 0)),
            pl.BlockSpec((_TB, 1), lambda h, b: (b, 0)),
            wspec, wspec, wspec, wspec,
        ],
        out_specs=pl.BlockSpec((1, _TB, EMB), lambda h, b: (h, b, 0)),
        out_shape=jax.ShapeDtypeStruct((HEADS, B, EMB), jnp.float32),
    )(neigh4, hsum, hl2, wqT, wk, wkT, wvT)


_TB2 = 512


def _t2_body(hout_ref, irow_ref, gi_ref, gu_ref, du_ref, di_ref, bu_ref, bi_ref,
             woT_ref, guT_ref, gub_ref, giT_ref, gib_ref,
             l1T_ref, l1b_ref, l2T_ref, l2b_ref, l3w_ref, l3b_ref, out_ref):
    ue = jnp.zeros((_TB2, EMB), jnp.float32)
    for h in range(HEADS):
        ue = ue + jnp.dot(hout_ref[h], woT_ref[h * EMB:(h + 1) * EMB, :],
                          preferred_element_type=jnp.float32)
    ie = irow_ref[...]
    item_din = du_ref[:, 0:1] + 1.0
    user_din = di_ref[:, 0:1] + 1.0
    gih = gi_ref[...] / item_din
    guh = gu_ref[...] / user_din
    go_u = jnp.maximum(
        jnp.dot(guh, guT_ref[...], preferred_element_type=jnp.float32)
        + gub_ref[...], 0.0)
    go_i = jnp.maximum(
        jnp.dot(gih, giT_ref[...], preferred_element_type=jnp.float32)
        + gib_ref[...], 0.0)
    xx = jnp.concatenate([ue * ie, ue * go_i, go_u * ie, go_u * go_i], axis=1)
    x1 = jnp.tanh(jnp.dot(xx, l1T_ref[...], preferred_element_type=jnp.float32)
                  + l1b_ref[...])
    x2 = jnp.tanh(jnp.dot(x1, l2T_ref[...], preferred_element_type=jnp.float32)
                  + l2b_ref[...])
    x3 = jnp.sum(x2 * l3w_ref[...], axis=1, keepdims=True) + l3b_ref[...]
    out_ref[...] = x3 + bu_ref[...] + bi_ref[...]


def _t2(hout, irows, gi, gu, du, di, bu2, bi2,
        woT, guT, gub2, giT, gib2, l1T, l1b2, l2T, l2b2, l3w2, l3b2):
    grid = (B // _TB2,)
    full = lambda *s: pl.BlockSpec(s, lambda i: tuple(0 for _ in s))
    return pl.pallas_call(
        _t2_body,
        grid=grid,
        in_specs=[
            pl.BlockSpec((HEADS, _TB2, EMB), lambda i: (0, i, 0)),
            pl.BlockSpec((_TB2, EMB), lambda i: (i, 0)),
            pl.BlockSpec((_TB2, EMB), lambda i: (i, 0)),
            pl.BlockSpec((_TB2, EMB), lambda i: (i, 0)),
            pl.BlockSpec((_TB2, 16), lambda i: (i, 0)),
            pl.BlockSpec((_TB2, 16), lambda i: (i, 0)),
            pl.BlockSpec((_TB2, 1), lambda i: (i, 0)),
            pl.BlockSpec((_TB2, 1), lambda i: (i, 0)),
            full(HEADS * EMB, EMB),
            full(EMB, EMB),
            full(1, EMB),
            full(EMB, EMB),
            full(1, EMB),
            full(4 * EMB, 128),
            full(1, 128),
            full(128, EMB),
            full(1, EMB),
            full(1, EMB),
            full(1, 1),
        ],
        out_specs=pl.BlockSpec((_TB2, 1), lambda i: (i, 0)),
        out_shape=jax.ShapeDtypeStruct((B, 1), jnp.float32),
    )(hout, irows, gi, gu, du, di, bu2, bi2,
      woT, guT, gub2, giT, gib2, l1T, l1b2, l2T, l2b2, l3w2, l3b2)


def kernel(x, history, history_len, supp_users, edge_sparse, user_embedding,
           item_embedding, wq, wk, wv, w_out_W, gcn_user_W, gcn_user_b,
           gcn_item_W, gcn_item_b, l1_W, l1_b, l2_W, l2_b, l3_W, l3_b,
           user_bias, item_bias):
    uid = x[:, 0]
    iid = x[:, 1]
    histp = jnp.pad(history, ((0, 0), (0, HIST_P - HIST)))
    suppp = jnp.pad(supp_users, (0, SUPP_P - SUPP))
    skey = jax.random.key(42)
    sis = [jax.random.randint(jax.random.fold_in(skey, i), (B, SAMPLE), 0, SUPP)
           for i in range(HEADS)]
    si_flat = jnp.stack(sis).reshape(-1)
    srcp = jnp.pad(edge_sparse[0], (0, EPAD - E), constant_values=INV_N - 1)
    dstp = jnp.pad(edge_sparse[1], (0, EPAD - E), constant_values=INV_N - 1)
    ub = user_bias.reshape(-1)
    ib = item_bias.reshape(-1)

    _inv_call, _supp_call, _gather_call, _edge_call, _bg_call = _sc_calls()
    invu, invi = _inv_call(uid, iid)
    esupp = _supp_call(suppp, user_embedding)
    neigh, hsum, irows = _gather_call(si_flat, histp.reshape(-1), iid, esupp,
                                      item_embedding)
    accu, acci, cntu, cnti = _edge_call(srcp, dstp, invu, invi,
                                        item_embedding, user_embedding)
    gi, gu, du, di, bu, bi = _bg_call(uid, iid, invu, invi,
                                      accu[0], accu[1], acci[0], acci[1],
                                      cntu[0], cntu[1], cnti[0], cnti[1],
                                      ub, ib)

    hl2 = history_len.astype(jnp.float32).reshape(B, 1)
    hout = _t1(neigh.reshape(HEADS, B, SAMPLE, EMB), hsum, hl2,
               wq.transpose(0, 2, 1), wk, wk.transpose(0, 2, 1),
               wv.transpose(0, 2, 1))
    out2 = _t2(hout, irows, gi, gu, du, di, bu.reshape(B, 1), bi.reshape(B, 1),
               w_out_W.T, gcn_user_W.T, gcn_user_b.reshape(1, EMB),
               gcn_item_W.T, gcn_item_b.reshape(1, EMB),
               l1_W.T, l1_b.reshape(1, -1), l2_W.T, l2_b.reshape(1, -1),
               l3_W.reshape(1, -1), l3_b.reshape(1, 1))
    return out2.reshape(-1)
